# baseline (speedup 1.0000x reference)
import functools

import numpy as np

import jax
import jax.numpy as jnp
from jax import lax
from jax.experimental import pallas as pl
from jax.experimental.pallas import tpu as pltpu
from jax.experimental.pallas import tpu_sc as plsc

N = 10000
E = 320000
D = 128
KE = 2
H = 16
C = 10

NC = 2
NS = 16
NW = NC * NS
EPW = E // NW
CHUNK = 1000
NCHUNK = EPW // CHUNK
N_PAD = 10240
STRIPE = N_PAD // NS

_mesh = plsc.VectorSubcoreMesh(core_axis_name="c", subcore_axis_name="s")

_ONES_T = np.ones((CHUNK, H), np.float32)
_ZEROS_T = np.zeros((STRIPE, H), np.float32)


def _make_edge_scatter(gather: bool):
    def body(src_hbm, dst_hbm, table_hbm, zeros_hbm, out_hbm,
             idxs0, idxs1, idxd0, idxd1, rows0, rows1, acc_sh,
             sem0, sem1, ssem0, ssem1):
        cid = lax.axis_index("c")
        sid = lax.axis_index("s")
        wid = cid * NS + sid
        idxs = [idxs0, idxs1]
        idxd = [idxd0, idxd1]
        rows = [rows0, rows1]
        sems = [sem0, sem1]
        ssems = [ssem0, ssem1]
        cps = [None, None]
        scps = [None, None]
        pltpu.sync_copy(zeros_hbm, acc_sh.at[pl.ds(sid * STRIPE, STRIPE)])
        if gather:
            pltpu.sync_copy(src_hbm.at[pl.ds(wid * EPW, CHUNK)], idxs0)
            cps[0] = pltpu.async_copy(table_hbm.at[idxs0], rows0, sem0)
            pltpu.sync_copy(dst_hbm.at[pl.ds(wid * EPW, CHUNK)], idxd0)
        else:
            pltpu.sync_copy(table_hbm, rows0)
        plsc.subcore_barrier()
        for k in range(NCHUNK):
            base = wid * EPW + k * CHUNK
            cur = k % 2
            if gather:
                if k + 1 < NCHUNK:
                    nxt = (k + 1) % 2
                    pltpu.sync_copy(src_hbm.at[pl.ds(base + CHUNK, CHUNK)],
                                    idxs[nxt])
                    if scps[nxt] is not None:
                        scps[nxt].wait()
                        scps[nxt] = None
                    cps[nxt] = pltpu.async_copy(table_hbm.at[idxs[nxt]],
                                                rows[nxt], sems[nxt])
                    pltpu.sync_copy(dst_hbm.at[pl.ds(base + CHUNK, CHUNK)],
                                    idxd[nxt])
                cps[cur].wait()
                scps[cur] = pltpu.async_copy(rows[cur], acc_sh.at[idxd[cur]],
                                             ssems[cur], add=True)
            else:
                pltpu.sync_copy(dst_hbm.at[pl.ds(base, CHUNK)], idxd0)
                pltpu.sync_copy(rows0, acc_sh.at[idxd0], add=True)
        for b in range(2):
            if scps[b] is not None:
                scps[b].wait()
        plsc.subcore_barrier()
        pltpu.sync_copy(acc_sh.at[pl.ds(sid * STRIPE, STRIPE)],
                        out_hbm.at[cid, pl.ds(sid * STRIPE, STRIPE)])

    return pl.kernel(
        body,
        mesh=_mesh,
        compiler_params=pltpu.CompilerParams(use_tc_tiling_on_sc=False),
        out_type=jax.ShapeDtypeStruct((NC, N_PAD, H), jnp.float32),
        scratch_types=[
            pltpu.VMEM((CHUNK,), jnp.int32),
            pltpu.VMEM((CHUNK,), jnp.int32),
            pltpu.VMEM((CHUNK,), jnp.int32),
            pltpu.VMEM((CHUNK,), jnp.int32),
            pltpu.VMEM((CHUNK, H), jnp.float32),
            pltpu.VMEM((CHUNK, H), jnp.float32),
            pltpu.VMEM_SHARED((N_PAD, H), jnp.float32),
            pltpu.SemaphoreType.DMA,
            pltpu.SemaphoreType.DMA,
            pltpu.SemaphoreType.DMA,
            pltpu.SemaphoreType.DMA,
        ],
    )


_deg_scatter = _make_edge_scatter(gather=False)
_edge_scatter = _make_edge_scatter(gather=True)


RW = N // 8
RWP = N_PAD // 8
GRP = 128 // H

_LANE = np.arange(128)
_P_BCAST = ((_LANE[:, None] % H == 0)
            & (_LANE[:, None] // H == _LANE[None, :] // H)).astype(np.float32)
_G_SUM = (_LANE[:, None] // H == _LANE[None, :] // H).astype(np.float32)


def _tc1a_body(xg_ref, keg_ref, w1blk_ref, keb_ref, hraw_ref):
    h = jnp.dot(xg_ref[...], w1blk_ref[...],
                preferred_element_type=jnp.float32)
    h = h + jnp.dot(keg_ref[...], keb_ref[...],
                    preferred_element_type=jnp.float32)
    hraw_ref[...] = h


_tc1a = pl.pallas_call(
    _tc1a_body,
    out_shape=jax.ShapeDtypeStruct((RW, 128), jnp.float32),
)


def _tc1b_body(deg_ref, hraw_ref, g1_ref, dinv_ref):
    deg = 1.0 + deg_ref[0, :RW] + deg_ref[1, :RW]
    dinv = lax.rsqrt(deg)
    g1_ref[...] = hraw_ref[...] * dinv
    dinv_ref[...] = dinv


_tc1b = pl.pallas_call(
    _tc1b_body,
    out_shape=(jax.ShapeDtypeStruct((RW, 128), jnp.float32),
               jax.ShapeDtypeStruct((RW, 128), jnp.float32)),
)


def _tc2_body(acc_ref, g1_ref, dinv_ref, b1_ref, w2blk_ref, g2_ref):
    acc = acc_ref[0, :RW] + acc_ref[1, :RW] + g1_ref[...]
    h1 = jnp.maximum(acc * dinv_ref[...] + b1_ref[...], 0.0)
    g2_ref[...] = jnp.dot(h1, w2blk_ref[...],
                          preferred_element_type=jnp.float32) * dinv_ref[...]


_tc2 = pl.pallas_call(
    _tc2_body,
    out_shape=jax.ShapeDtypeStruct((RW, 128), jnp.float32),
)


def _tc3_body(acc_ref, g2_ref, dinv_ref, b2_ref, p_ref, g_ref, out_ref):
    t = (acc_ref[0, :RW] + acc_ref[1, :RW] + g2_ref[...]) * dinv_ref[...]
    t = t + b2_ref[...]
    lane = lax.broadcasted_iota(jnp.int32, (RW, 128), 1)
    jm = lane % H
    valid = jm < C
    tm = jnp.where(valid, t, -3.0e38)
    for s in (1, 2, 4, 8):
        rolled = jnp.concatenate([tm[:, s:], tm[:, :s]], axis=1)
        keep = (jm + s) < H
        tm = jnp.where(keep, jnp.maximum(tm, rolled), tm)
    mb = jnp.dot(tm, p_ref[...], preferred_element_type=jnp.float32)
    ex = jnp.where(valid, jnp.exp(t - mb), 0.0)
    ssum = jnp.dot(ex, g_ref[...], preferred_element_type=jnp.float32)
    out_ref[...] = t - mb - jnp.log(ssum)


_tc3 = pl.pallas_call(
    _tc3_body,
    out_shape=jax.ShapeDtypeStruct((RW, 128), jnp.float32),
)


def kernel(x, edge_index, ke, W1, b1, W2, b2):
    src = edge_index[0]
    dst = edge_index[1]
    ones_t = jnp.asarray(_ONES_T)
    zeros_t = jnp.asarray(_ZEROS_T)
    eye8 = jnp.eye(GRP, dtype=jnp.float32)
    W1blk = jnp.kron(eye8, W1[:D])
    KEb = jnp.kron(eye8, W1[D:])
    W2p = jnp.concatenate([W2, jnp.zeros((H, H - C), W2.dtype)], axis=1)
    W2blk = jnp.kron(eye8, W2p)
    b1t = jnp.tile(b1, GRP)
    b2t = jnp.tile(jnp.concatenate([b2, jnp.zeros((H - C,), b2.dtype)]), GRP)
    xg = x.reshape(RW, GRP * D)
    keg = ke.reshape(RW, GRP * KE)

    deg_parts = _deg_scatter(dst, dst, ones_t, zeros_t)
    hraw = _tc1a(xg, keg, W1blk, KEb)
    g1w, dinvw = _tc1b(deg_parts.reshape(NC, RWP, 128), hraw)
    acc1 = _edge_scatter(src, dst, g1w.reshape(N, H), zeros_t)
    g2w = _tc2(acc1.reshape(NC, RWP, 128), g1w, dinvw, b1t, W2blk)
    acc2 = _edge_scatter(src, dst, g2w.reshape(N, H), zeros_t)
    outw = _tc3(acc2.reshape(NC, RWP, 128), g2w, dinvw, b2t,
                jnp.asarray(_P_BCAST), jnp.asarray(_G_SUM))
    return outw.reshape(RW, GRP, H)[:, :, :C].reshape(N, C)


# speedup vs baseline: 1.0296x; 1.0296x over previous
import functools

import numpy as np

import jax
import jax.numpy as jnp
from jax import lax
from jax.experimental import pallas as pl
from jax.experimental.pallas import tpu as pltpu
from jax.experimental.pallas import tpu_sc as plsc

N = 10000
E = 320000
D = 128
KE = 2
H = 16
C = 10

NC = 2
NS = 16
NW = NC * NS
EPW = E // NW
CHUNK = 2000
NCHUNK = EPW // CHUNK
N_PAD = 10240
STRIPE = N_PAD // NS

_mesh = plsc.VectorSubcoreMesh(core_axis_name="c", subcore_axis_name="s")

_ONES_T = np.ones((CHUNK, H), np.float32)
_ZEROS_T = np.zeros((STRIPE, H), np.float32)


def _make_edge_scatter(gather: bool):
    def body(src_hbm, dst_hbm, table_hbm, zeros_hbm, out_hbm,
             idxs0, idxs1, idxd0, idxd1, rows0, rows1, acc_sh,
             sem0, sem1, ssem0, ssem1):
        cid = lax.axis_index("c")
        sid = lax.axis_index("s")
        wid = cid * NS + sid
        idxs = [idxs0, idxs1]
        idxd = [idxd0, idxd1]
        rows = [rows0, rows1]
        sems = [sem0, sem1]
        ssems = [ssem0, ssem1]
        cps = [None, None]
        scps = [None, None]
        pltpu.sync_copy(zeros_hbm, acc_sh.at[pl.ds(sid * STRIPE, STRIPE)])
        if gather:
            pltpu.sync_copy(src_hbm.at[pl.ds(wid * EPW, CHUNK)], idxs0)
            cps[0] = pltpu.async_copy(table_hbm.at[idxs0], rows0, sem0)
            pltpu.sync_copy(dst_hbm.at[pl.ds(wid * EPW, CHUNK)], idxd0)
        else:
            pltpu.sync_copy(table_hbm, rows0)
        plsc.subcore_barrier()
        for k in range(NCHUNK):
            base = wid * EPW + k * CHUNK
            cur = k % 2
            if gather:
                if k + 1 < NCHUNK:
                    nxt = (k + 1) % 2
                    pltpu.sync_copy(src_hbm.at[pl.ds(base + CHUNK, CHUNK)],
                                    idxs[nxt])
                    if scps[nxt] is not None:
                        scps[nxt].wait()
                        scps[nxt] = None
                    cps[nxt] = pltpu.async_copy(table_hbm.at[idxs[nxt]],
                                                rows[nxt], sems[nxt])
                    pltpu.sync_copy(dst_hbm.at[pl.ds(base + CHUNK, CHUNK)],
                                    idxd[nxt])
                cps[cur].wait()
                scps[cur] = pltpu.async_copy(rows[cur], acc_sh.at[idxd[cur]],
                                             ssems[cur], add=True)
            else:
                pltpu.sync_copy(dst_hbm.at[pl.ds(base, CHUNK)], idxd0)
                pltpu.sync_copy(rows0, acc_sh.at[idxd0], add=True)
        for b in range(2):
            if scps[b] is not None:
                scps[b].wait()
        plsc.subcore_barrier()
        pltpu.sync_copy(acc_sh.at[pl.ds(sid * STRIPE, STRIPE)],
                        out_hbm.at[cid, pl.ds(sid * STRIPE, STRIPE)])

    return pl.kernel(
        body,
        mesh=_mesh,
        compiler_params=pltpu.CompilerParams(use_tc_tiling_on_sc=False),
        out_type=jax.ShapeDtypeStruct((NC, N_PAD, H), jnp.float32),
        scratch_types=[
            pltpu.VMEM((CHUNK,), jnp.int32),
            pltpu.VMEM((CHUNK,), jnp.int32),
            pltpu.VMEM((CHUNK,), jnp.int32),
            pltpu.VMEM((CHUNK,), jnp.int32),
            pltpu.VMEM((CHUNK, H), jnp.float32),
            pltpu.VMEM((CHUNK, H), jnp.float32),
            pltpu.VMEM_SHARED((N_PAD, H), jnp.float32),
            pltpu.SemaphoreType.DMA,
            pltpu.SemaphoreType.DMA,
            pltpu.SemaphoreType.DMA,
            pltpu.SemaphoreType.DMA,
        ],
    )


_deg_scatter = _make_edge_scatter(gather=False)
_edge_scatter = _make_edge_scatter(gather=True)


RW = N // 8
RWP = N_PAD // 8
GRP = 128 // H

_LANE = np.arange(128)
_P_BCAST = ((_LANE[:, None] % H == 0)
            & (_LANE[:, None] // H == _LANE[None, :] // H)).astype(np.float32)
_G_SUM = (_LANE[:, None] // H == _LANE[None, :] // H).astype(np.float32)


def _tc1a_body(xg_ref, keg_ref, w1blk_ref, keb_ref, hraw_ref):
    h = jnp.dot(xg_ref[...], w1blk_ref[...],
                preferred_element_type=jnp.float32)
    h = h + jnp.dot(keg_ref[...], keb_ref[...],
                    preferred_element_type=jnp.float32)
    hraw_ref[...] = h


_tc1a = pl.pallas_call(
    _tc1a_body,
    out_shape=jax.ShapeDtypeStruct((RW, 128), jnp.float32),
)


def _tc1b_body(deg_ref, hraw_ref, g1_ref, dinv_ref):
    deg = 1.0 + deg_ref[0, :RW] + deg_ref[1, :RW]
    dinv = lax.rsqrt(deg)
    g1_ref[...] = hraw_ref[...] * dinv
    dinv_ref[...] = dinv


_tc1b = pl.pallas_call(
    _tc1b_body,
    out_shape=(jax.ShapeDtypeStruct((RW, 128), jnp.float32),
               jax.ShapeDtypeStruct((RW, 128), jnp.float32)),
)


def _tc2_body(acc_ref, g1_ref, dinv_ref, b1_ref, w2blk_ref, g2_ref):
    acc = acc_ref[0, :RW] + acc_ref[1, :RW] + g1_ref[...]
    h1 = jnp.maximum(acc * dinv_ref[...] + b1_ref[...], 0.0)
    g2_ref[...] = jnp.dot(h1, w2blk_ref[...],
                          preferred_element_type=jnp.float32) * dinv_ref[...]


_tc2 = pl.pallas_call(
    _tc2_body,
    out_shape=jax.ShapeDtypeStruct((RW, 128), jnp.float32),
)


def _tc3_body(acc_ref, g2_ref, dinv_ref, b2_ref, p_ref, g_ref, out_ref):
    t = (acc_ref[0, :RW] + acc_ref[1, :RW] + g2_ref[...]) * dinv_ref[...]
    t = t + b2_ref[...]
    lane = lax.broadcasted_iota(jnp.int32, (RW, 128), 1)
    jm = lane % H
    valid = jm < C
    tm = jnp.where(valid, t, -3.0e38)
    for s in (1, 2, 4, 8):
        rolled = jnp.concatenate([tm[:, s:], tm[:, :s]], axis=1)
        keep = (jm + s) < H
        tm = jnp.where(keep, jnp.maximum(tm, rolled), tm)
    mb = jnp.dot(tm, p_ref[...], preferred_element_type=jnp.float32)
    ex = jnp.where(valid, jnp.exp(t - mb), 0.0)
    ssum = jnp.dot(ex, g_ref[...], preferred_element_type=jnp.float32)
    out_ref[...] = t - mb - jnp.log(ssum)


_tc3 = pl.pallas_call(
    _tc3_body,
    out_shape=jax.ShapeDtypeStruct((RW, 128), jnp.float32),
)


def kernel(x, edge_index, ke, W1, b1, W2, b2):
    src = edge_index[0]
    dst = edge_index[1]
    ones_t = jnp.asarray(_ONES_T)
    zeros_t = jnp.asarray(_ZEROS_T)
    eye8 = jnp.eye(GRP, dtype=jnp.float32)
    W1blk = jnp.kron(eye8, W1[:D])
    KEb = jnp.kron(eye8, W1[D:])
    W2p = jnp.concatenate([W2, jnp.zeros((H, H - C), W2.dtype)], axis=1)
    W2blk = jnp.kron(eye8, W2p)
    b1t = jnp.tile(b1, GRP)
    b2t = jnp.tile(jnp.concatenate([b2, jnp.zeros((H - C,), b2.dtype)]), GRP)
    xg = x.reshape(RW, GRP * D)
    keg = ke.reshape(RW, GRP * KE)

    deg_parts = _deg_scatter(dst, dst, ones_t, zeros_t)
    hraw = _tc1a(xg, keg, W1blk, KEb)
    g1w, dinvw = _tc1b(deg_parts.reshape(NC, RWP, 128), hraw)
    acc1 = _edge_scatter(src, dst, g1w.reshape(N, H), zeros_t)
    g2w = _tc2(acc1.reshape(NC, RWP, 128), g1w, dinvw, b1t, W2blk)
    acc2 = _edge_scatter(src, dst, g2w.reshape(N, H), zeros_t)
    outw = _tc3(acc2.reshape(NC, RWP, 128), g2w, dinvw, b2t,
                jnp.asarray(_P_BCAST), jnp.asarray(_G_SUM))
    return outw.reshape(RW, GRP, H)[:, :, :C].reshape(N, C)


# speedup vs baseline: 1.1640x; 1.1305x over previous
import functools

import numpy as np

import jax
import jax.numpy as jnp
from jax import lax
from jax.experimental import pallas as pl
from jax.experimental.pallas import tpu as pltpu
from jax.experimental.pallas import tpu_sc as plsc

N = 10000
E = 320000
D = 128
KE = 2
H = 16
C = 10

NC = 2
NS = 16
NW = NC * NS
EPW = E // NW
CHUNK = 2000
NCHUNK = EPW // CHUNK
N_PAD = 10240
STRIPE = N_PAD // NS

_mesh = plsc.VectorSubcoreMesh(core_axis_name="c", subcore_axis_name="s")

_ONES_T = np.ones((CHUNK, H), np.float32)
_ZEROS_T = np.zeros((STRIPE, H), np.float32)


def _make_edge_scatter(gather: bool):
    def body(src_hbm, dst_hbm, table_hbm, zeros_hbm, out_hbm,
             idxs0, idxs1, idxd0, idxd1, rows0, rows1, acc_sh,
             sem0, sem1, ssem0, ssem1):
        cid = lax.axis_index("c")
        sid = lax.axis_index("s")
        wid = cid * NS + sid
        idxs = [idxs0, idxs1]
        idxd = [idxd0, idxd1]
        rows = [rows0, rows1]
        sems = [sem0, sem1]
        ssems = [ssem0, ssem1]
        cps = [None, None]
        scps = [None, None]
        pltpu.sync_copy(zeros_hbm, acc_sh.at[pl.ds(sid * STRIPE, STRIPE)])
        if gather:
            pltpu.sync_copy(src_hbm.at[pl.ds(wid * EPW, CHUNK)], idxs0)
            cps[0] = pltpu.async_copy(table_hbm.at[idxs0], rows0, sem0)
            pltpu.sync_copy(dst_hbm.at[pl.ds(wid * EPW, CHUNK)], idxd0)
        else:
            pltpu.sync_copy(table_hbm, rows0)
        plsc.subcore_barrier()
        for k in range(NCHUNK):
            base = wid * EPW + k * CHUNK
            cur = k % 2
            if gather:
                if k + 1 < NCHUNK:
                    nxt = (k + 1) % 2
                    pltpu.sync_copy(src_hbm.at[pl.ds(base + CHUNK, CHUNK)],
                                    idxs[nxt])
                    if scps[nxt] is not None:
                        scps[nxt].wait()
                        scps[nxt] = None
                    cps[nxt] = pltpu.async_copy(table_hbm.at[idxs[nxt]],
                                                rows[nxt], sems[nxt])
                    pltpu.sync_copy(dst_hbm.at[pl.ds(base + CHUNK, CHUNK)],
                                    idxd[nxt])
                cps[cur].wait()
                scps[cur] = pltpu.async_copy(rows[cur], acc_sh.at[idxd[cur]],
                                             ssems[cur], add=True)
            else:
                pltpu.sync_copy(dst_hbm.at[pl.ds(base, CHUNK)], idxd0)
                pltpu.sync_copy(rows0, acc_sh.at[idxd0], add=True)
        for b in range(2):
            if scps[b] is not None:
                scps[b].wait()
        plsc.subcore_barrier()
        pltpu.sync_copy(acc_sh.at[pl.ds(sid * STRIPE, STRIPE)],
                        out_hbm.at[cid, pl.ds(sid * STRIPE, STRIPE)])

    return pl.kernel(
        body,
        mesh=_mesh,
        compiler_params=pltpu.CompilerParams(use_tc_tiling_on_sc=False),
        out_type=jax.ShapeDtypeStruct((NC, N_PAD, H), jnp.float32),
        scratch_types=[
            pltpu.VMEM((CHUNK,), jnp.int32),
            pltpu.VMEM((CHUNK,), jnp.int32),
            pltpu.VMEM((CHUNK,), jnp.int32),
            pltpu.VMEM((CHUNK,), jnp.int32),
            pltpu.VMEM((CHUNK, H), jnp.float32),
            pltpu.VMEM((CHUNK, H), jnp.float32),
            pltpu.VMEM_SHARED((N_PAD, H), jnp.float32),
            pltpu.SemaphoreType.DMA,
            pltpu.SemaphoreType.DMA,
            pltpu.SemaphoreType.DMA,
            pltpu.SemaphoreType.DMA,
        ],
    )


_deg_scatter = _make_edge_scatter(gather=False)
_edge_scatter = _make_edge_scatter(gather=True)


RW = N // 8
RWP = N_PAD // 8
GRP = 128 // H

_LANE = np.arange(128)
_P_BCAST = ((_LANE[:, None] % H == 0)
            & (_LANE[:, None] // H == _LANE[None, :] // H)).astype(np.float32)
_G_SUM = (_LANE[:, None] // H == _LANE[None, :] // H).astype(np.float32)


_SPB = E // 10


def _split_body(e_ref, s_ref, d_ref):
    s_ref[...] = e_ref[0].reshape(1, _SPB // 128, 128)
    d_ref[...] = e_ref[1].reshape(1, _SPB // 128, 128)


_tcsplit = pl.pallas_call(
    _split_body,
    grid=(10,),
    in_specs=[pl.BlockSpec((2, _SPB), lambda i: (0, i))],
    out_specs=(pl.BlockSpec((1, _SPB // 128, 128), lambda i: (i, 0, 0)),
               pl.BlockSpec((1, _SPB // 128, 128), lambda i: (i, 0, 0))),
    out_shape=(jax.ShapeDtypeStruct((10, _SPB // 128, 128), jnp.int32),
               jax.ShapeDtypeStruct((10, _SPB // 128, 128), jnp.int32)),
)


def _tc1a_body(x3_ref, keg_ref, w1a_ref, keb_ref, hraw_ref):
    h3 = lax.dot_general(x3_ref[...], w1a_ref[...],
                         dimension_numbers=(((2,), (0,)), ((), ())),
                         preferred_element_type=jnp.float32)
    h = h3.reshape(RW, 128)
    h = h + jnp.dot(keg_ref[...], keb_ref[...],
                    preferred_element_type=jnp.float32)
    hraw_ref[...] = h


_tc1a = pl.pallas_call(
    _tc1a_body,
    out_shape=jax.ShapeDtypeStruct((RW, 128), jnp.float32),
)


def _tc1b_body(deg_ref, hraw_ref, g1_ref, dinv_ref):
    deg = 1.0 + deg_ref[0, :RW] + deg_ref[1, :RW]
    dinv = lax.rsqrt(deg)
    g1_ref[...] = hraw_ref[...] * dinv
    dinv_ref[...] = dinv


_tc1b = pl.pallas_call(
    _tc1b_body,
    out_shape=(jax.ShapeDtypeStruct((RW, 128), jnp.float32),
               jax.ShapeDtypeStruct((RW, 128), jnp.float32)),
)


def _tc2_body(acc_ref, g1_ref, dinv_ref, b1_ref, w2blk_ref, g2_ref):
    acc = acc_ref[0, :RW] + acc_ref[1, :RW] + g1_ref[...]
    h1 = jnp.maximum(acc * dinv_ref[...] + b1_ref[...], 0.0)
    g2_ref[...] = jnp.dot(h1, w2blk_ref[...],
                          preferred_element_type=jnp.float32) * dinv_ref[...]


_tc2 = pl.pallas_call(
    _tc2_body,
    out_shape=jax.ShapeDtypeStruct((RW, 128), jnp.float32),
)


def _tc3_body(acc_ref, g2_ref, dinv_ref, b2_ref, p_ref, g_ref, out_ref):
    t = (acc_ref[0, :RW] + acc_ref[1, :RW] + g2_ref[...]) * dinv_ref[...]
    t = t + b2_ref[...]
    lane = lax.broadcasted_iota(jnp.int32, (RW, 128), 1)
    jm = lane % H
    valid = jm < C
    tm = jnp.where(valid, t, -3.0e38)
    for s in (1, 2, 4, 8):
        rolled = jnp.concatenate([tm[:, s:], tm[:, :s]], axis=1)
        keep = (jm + s) < H
        tm = jnp.where(keep, jnp.maximum(tm, rolled), tm)
    mb = jnp.dot(tm, p_ref[...], preferred_element_type=jnp.float32)
    ex = jnp.where(valid, jnp.exp(t - mb), 0.0)
    ssum = jnp.dot(ex, g_ref[...], preferred_element_type=jnp.float32)
    out_ref[...] = t - mb - jnp.log(ssum)


_tc3 = pl.pallas_call(
    _tc3_body,
    out_shape=jax.ShapeDtypeStruct((RW, 128), jnp.float32),
)


def kernel(x, edge_index, ke, W1, b1, W2, b2):
    src2, dst2 = _tcsplit(edge_index)
    src = src2.reshape(E)
    dst = dst2.reshape(E)
    ones_t = jnp.asarray(_ONES_T)
    zeros_t = jnp.asarray(_ZEROS_T)
    eye8 = jnp.eye(GRP, dtype=jnp.float32)
    KEb = jnp.kron(eye8, W1[D:])
    W2p = jnp.concatenate([W2, jnp.zeros((H, H - C), W2.dtype)], axis=1)
    W2blk = jnp.kron(eye8, W2p)
    b1t = jnp.tile(b1, GRP)
    b2t = jnp.tile(jnp.concatenate([b2, jnp.zeros((H - C,), b2.dtype)]), GRP)
    x3 = x.reshape(RW, GRP, D)
    keg = ke.reshape(RW, GRP * KE)

    deg_parts = _deg_scatter(dst, dst, ones_t, zeros_t)
    hraw = _tc1a(x3, keg, W1[:D], KEb)
    g1w, dinvw = _tc1b(deg_parts.reshape(NC, RWP, 128), hraw)
    acc1 = _edge_scatter(src, dst, g1w.reshape(N, H), zeros_t)
    g2w = _tc2(acc1.reshape(NC, RWP, 128), g1w, dinvw, b1t, W2blk)
    acc2 = _edge_scatter(src, dst, g2w.reshape(N, H), zeros_t)
    outw = _tc3(acc2.reshape(NC, RWP, 128), g2w, dinvw, b2t,
                jnp.asarray(_P_BCAST), jnp.asarray(_G_SUM))
    return outw.reshape(N, H)[:, :C]


# speedup vs baseline: 1.2083x; 1.0381x over previous
import functools

import numpy as np

import jax
import jax.numpy as jnp
from jax import lax
from jax.experimental import pallas as pl
from jax.experimental.pallas import tpu as pltpu
from jax.experimental.pallas import tpu_sc as plsc

N = 10000
E = 320000
D = 128
KE = 2
H = 16
C = 10

NC = 2
NS = 16
NW = NC * NS
EPW = E // NW
CHUNK = 2000
NCHUNK = EPW // CHUNK
N_PAD = 10240
STRIPE = N_PAD // NS

_mesh = plsc.VectorSubcoreMesh(core_axis_name="c", subcore_axis_name="s")

_ONES_T = np.ones((CHUNK, H), np.float32)
_ZEROS_T = np.zeros((STRIPE, H), np.float32)
_ONES_N = np.ones((CHUNK,), np.float32)
_ZEROS_N = np.zeros((STRIPE,), np.float32)


def _deg_body(dst_hbm, ones_hbm, zeros_hbm, out_hbm,
              idxd0, idxd1, ones_v, dn_v, rep_v, deg_sh, ssem0, ssem1):
    cid = lax.axis_index("c")
    sid = lax.axis_index("s")
    wid = cid * NS + sid
    idxd = [idxd0, idxd1]
    ssems = [ssem0, ssem1]
    scps = [None, None]
    pltpu.sync_copy(zeros_hbm, deg_sh.at[pl.ds(sid * STRIPE, STRIPE)])
    pltpu.sync_copy(ones_hbm, ones_v)
    pltpu.sync_copy(dst_hbm.at[pl.ds(wid * EPW, CHUNK)], idxd0)
    plsc.subcore_barrier()
    for k in range(NCHUNK):
        cur = k % 2
        if k + 1 < NCHUNK:
            nxt = (k + 1) % 2
            if scps[nxt] is not None:
                scps[nxt].wait()
                scps[nxt] = None
            pltpu.sync_copy(
                dst_hbm.at[pl.ds(wid * EPW + (k + 1) * CHUNK, CHUNK)],
                idxd[nxt])
        scps[cur] = pltpu.async_copy(ones_v, deg_sh.at[idxd[cur]],
                                     ssems[cur], add=True)
    for b in range(2):
        if scps[b] is not None:
            scps[b].wait()
    plsc.subcore_barrier()
    pltpu.sync_copy(deg_sh.at[pl.ds(sid * STRIPE, STRIPE)], dn_v)

    def rep_row(i, _):
        chunk = dn_v[pl.ds(i * 16, 16)]
        for m in range(16):
            rep_v[i * 16 + m] = jnp.full((16,), chunk[m], jnp.float32)
        return 0

    lax.fori_loop(0, STRIPE // 16, rep_row, 0)
    pltpu.sync_copy(rep_v, out_hbm.at[cid, pl.ds(sid * STRIPE, STRIPE)])


_deg_scatter = pl.kernel(
    _deg_body,
    mesh=_mesh,
    compiler_params=pltpu.CompilerParams(use_tc_tiling_on_sc=False),
    out_type=jax.ShapeDtypeStruct((NC, N_PAD, H), jnp.float32),
    scratch_types=[
        pltpu.VMEM((CHUNK,), jnp.int32),
        pltpu.VMEM((CHUNK,), jnp.int32),
        pltpu.VMEM((CHUNK,), jnp.float32),
        pltpu.VMEM((STRIPE,), jnp.float32),
        pltpu.VMEM((STRIPE, H), jnp.float32),
        pltpu.VMEM_SHARED((N_PAD,), jnp.float32),
        pltpu.SemaphoreType.DMA,
        pltpu.SemaphoreType.DMA,
    ],
)


def _make_edge_scatter(gather: bool):
    def body(src_hbm, dst_hbm, table_hbm, zeros_hbm, out_hbm,
             idxs0, idxs1, idxd0, idxd1, rows0, rows1, acc_sh,
             sem0, sem1, ssem0, ssem1):
        cid = lax.axis_index("c")
        sid = lax.axis_index("s")
        wid = cid * NS + sid
        idxs = [idxs0, idxs1]
        idxd = [idxd0, idxd1]
        rows = [rows0, rows1]
        sems = [sem0, sem1]
        ssems = [ssem0, ssem1]
        cps = [None, None]
        scps = [None, None]
        pltpu.sync_copy(zeros_hbm, acc_sh.at[pl.ds(sid * STRIPE, STRIPE)])
        if gather:
            pltpu.sync_copy(src_hbm.at[pl.ds(wid * EPW, CHUNK)], idxs0)
            cps[0] = pltpu.async_copy(table_hbm.at[idxs0], rows0, sem0)
            pltpu.sync_copy(dst_hbm.at[pl.ds(wid * EPW, CHUNK)], idxd0)
        else:
            pltpu.sync_copy(table_hbm, rows0)
        plsc.subcore_barrier()
        for k in range(NCHUNK):
            base = wid * EPW + k * CHUNK
            cur = k % 2
            if gather:
                if k + 1 < NCHUNK:
                    nxt = (k + 1) % 2
                    pltpu.sync_copy(src_hbm.at[pl.ds(base + CHUNK, CHUNK)],
                                    idxs[nxt])
                    if scps[nxt] is not None:
                        scps[nxt].wait()
                        scps[nxt] = None
                    cps[nxt] = pltpu.async_copy(table_hbm.at[idxs[nxt]],
                                                rows[nxt], sems[nxt])
                    pltpu.sync_copy(dst_hbm.at[pl.ds(base + CHUNK, CHUNK)],
                                    idxd[nxt])
                cps[cur].wait()
                scps[cur] = pltpu.async_copy(rows[cur], acc_sh.at[idxd[cur]],
                                             ssems[cur], add=True)
            else:
                pltpu.sync_copy(dst_hbm.at[pl.ds(base, CHUNK)], idxd0)
                pltpu.sync_copy(rows0, acc_sh.at[idxd0], add=True)
        for b in range(2):
            if scps[b] is not None:
                scps[b].wait()
        plsc.subcore_barrier()
        pltpu.sync_copy(acc_sh.at[pl.ds(sid * STRIPE, STRIPE)],
                        out_hbm.at[cid, pl.ds(sid * STRIPE, STRIPE)])

    return pl.kernel(
        body,
        mesh=_mesh,
        compiler_params=pltpu.CompilerParams(use_tc_tiling_on_sc=False),
        out_type=jax.ShapeDtypeStruct((NC, N_PAD, H), jnp.float32),
        scratch_types=[
            pltpu.VMEM((CHUNK,), jnp.int32),
            pltpu.VMEM((CHUNK,), jnp.int32),
            pltpu.VMEM((CHUNK,), jnp.int32),
            pltpu.VMEM((CHUNK,), jnp.int32),
            pltpu.VMEM((CHUNK, H), jnp.float32),
            pltpu.VMEM((CHUNK, H), jnp.float32),
            pltpu.VMEM_SHARED((N_PAD, H), jnp.float32),
            pltpu.SemaphoreType.DMA,
            pltpu.SemaphoreType.DMA,
            pltpu.SemaphoreType.DMA,
            pltpu.SemaphoreType.DMA,
        ],
    )


_edge_scatter = _make_edge_scatter(gather=True)


RW = N // 8
RWP = N_PAD // 8
GRP = 128 // H

_LANE = np.arange(128)
_P_BCAST = ((_LANE[:, None] % H == 0)
            & (_LANE[:, None] // H == _LANE[None, :] // H)).astype(np.float32)
_G_SUM = (_LANE[:, None] // H == _LANE[None, :] // H).astype(np.float32)


_SPB = E // 10


def _split_body(e_ref, s_ref, d_ref):
    s_ref[...] = e_ref[0].reshape(1, _SPB // 128, 128)
    d_ref[...] = e_ref[1].reshape(1, _SPB // 128, 128)


_tcsplit = pl.pallas_call(
    _split_body,
    grid=(10,),
    in_specs=[pl.BlockSpec((2, _SPB), lambda i: (0, i))],
    out_specs=(pl.BlockSpec((1, _SPB // 128, 128), lambda i: (i, 0, 0)),
               pl.BlockSpec((1, _SPB // 128, 128), lambda i: (i, 0, 0))),
    out_shape=(jax.ShapeDtypeStruct((10, _SPB // 128, 128), jnp.int32),
               jax.ShapeDtypeStruct((10, _SPB // 128, 128), jnp.int32)),
)


def _tc1a_body(x3_ref, ke3_ref, w1a_ref, w1b_ref, hraw_ref):
    h3 = lax.dot_general(x3_ref[...], w1a_ref[...],
                         dimension_numbers=(((2,), (0,)), ((), ())),
                         preferred_element_type=jnp.float32)
    k3 = lax.dot_general(ke3_ref[...], w1b_ref[...],
                         dimension_numbers=(((2,), (0,)), ((), ())),
                         preferred_element_type=jnp.float32)
    hraw_ref[...] = (h3 + k3).reshape(RW, 128)


_tc1a = pl.pallas_call(
    _tc1a_body,
    out_shape=jax.ShapeDtypeStruct((RW, 128), jnp.float32),
)


def _tc1b_body(deg_ref, hraw_ref, g1_ref, dinv_ref):
    deg = 1.0 + deg_ref[0, :RW] + deg_ref[1, :RW]
    dinv = lax.rsqrt(deg)
    g1_ref[...] = hraw_ref[...] * dinv
    dinv_ref[...] = dinv


_tc1b = pl.pallas_call(
    _tc1b_body,
    out_shape=(jax.ShapeDtypeStruct((RW, 128), jnp.float32),
               jax.ShapeDtypeStruct((RW, 128), jnp.float32)),
)


def _tc2_body(acc_ref, g1_ref, dinv_ref, b1_ref, w2blk_ref, g2_ref):
    acc = acc_ref[0, :RW] + acc_ref[1, :RW] + g1_ref[...]
    h1 = jnp.maximum(acc * dinv_ref[...] + b1_ref[...], 0.0)
    g2_ref[...] = jnp.dot(h1, w2blk_ref[...],
                          preferred_element_type=jnp.float32) * dinv_ref[...]


_tc2 = pl.pallas_call(
    _tc2_body,
    out_shape=jax.ShapeDtypeStruct((RW, 128), jnp.float32),
)


def _tc3_body(acc_ref, g2_ref, dinv_ref, b2_ref, p_ref, g_ref, out_ref):
    t = (acc_ref[0, :RW] + acc_ref[1, :RW] + g2_ref[...]) * dinv_ref[...]
    t = t + b2_ref[...]
    lane = lax.broadcasted_iota(jnp.int32, (RW, 128), 1)
    jm = lane % H
    valid = jm < C
    tm = jnp.where(valid, t, -3.0e38)
    for s in (1, 2, 4, 8):
        rolled = jnp.concatenate([tm[:, s:], tm[:, :s]], axis=1)
        keep = (jm + s) < H
        tm = jnp.where(keep, jnp.maximum(tm, rolled), tm)
    mb = jnp.dot(tm, p_ref[...], preferred_element_type=jnp.float32)
    ex = jnp.where(valid, jnp.exp(t - mb), 0.0)
    ssum = jnp.dot(ex, g_ref[...], preferred_element_type=jnp.float32)
    out_ref[...] = t - mb - jnp.log(ssum)


_tc3 = pl.pallas_call(
    _tc3_body,
    out_shape=jax.ShapeDtypeStruct((RW, 128), jnp.float32),
)


def kernel(x, edge_index, ke, W1, b1, W2, b2):
    src2, dst2 = _tcsplit(edge_index)
    src = src2.reshape(E)
    dst = dst2.reshape(E)
    ones_t = jnp.asarray(_ONES_T)
    zeros_t = jnp.asarray(_ZEROS_T)
    eye8 = jnp.eye(GRP, dtype=jnp.float32)
    W2p = jnp.concatenate([W2, jnp.zeros((H, H - C), W2.dtype)], axis=1)
    W2blk = jnp.kron(eye8, W2p)
    b1t = jnp.tile(b1, GRP)
    b2t = jnp.tile(jnp.concatenate([b2, jnp.zeros((H - C,), b2.dtype)]), GRP)
    x3 = x.reshape(RW, GRP, D)
    ke3 = ke.reshape(RW, GRP, KE)

    deg_parts = _deg_scatter(dst, jnp.asarray(_ONES_N), jnp.asarray(_ZEROS_N))
    hraw = _tc1a(x3, ke3, W1[:D], W1[D:])
    g1w, dinvw = _tc1b(deg_parts.reshape(NC, RWP, 128), hraw)
    acc1 = _edge_scatter(src, dst, g1w.reshape(N, H), zeros_t)
    g2w = _tc2(acc1.reshape(NC, RWP, 128), g1w, dinvw, b1t, W2blk)
    acc2 = _edge_scatter(src, dst, g2w.reshape(N, H), zeros_t)
    outw = _tc3(acc2.reshape(NC, RWP, 128), g2w, dinvw, b2t,
                jnp.asarray(_P_BCAST), jnp.asarray(_G_SUM))
    return outw.reshape(N, H)[:, :C]


# speedup vs baseline: 1.2102x; 1.0016x over previous
import numpy as np

import jax
import jax.numpy as jnp
from jax import lax
from jax.experimental import pallas as pl
from jax.experimental.pallas import tpu as pltpu
from jax.experimental.pallas import tpu_sc as plsc

N = 10000
E = 320000
D = 128
KE = 2
H = 16
C = 10

NC = 2
NS = 16
NW = NC * NS
EPW = E // NW
CHUNK = 2000
NCHUNK = EPW // CHUNK
N_PAD = 10240
STRIPE = N_PAD // NS

_mesh = plsc.VectorSubcoreMesh(core_axis_name="c", subcore_axis_name="s")

_ZEROS_T = np.zeros((STRIPE, H), np.float32)
_ONES_N = np.ones((CHUNK,), np.float32)
_ZEROS_N = np.zeros((STRIPE,), np.float32)


def _deg_body(dst_hbm, ones_hbm, zeros_hbm, out_hbm,
              idxd0, idxd1, ones_v, dn_v, rep_v, deg_sh, ssem0, ssem1):
    cid = lax.axis_index("c")
    sid = lax.axis_index("s")
    wid = cid * NS + sid
    idxd = [idxd0, idxd1]
    ssems = [ssem0, ssem1]
    scps = [None, None]
    pltpu.sync_copy(zeros_hbm, deg_sh.at[pl.ds(sid * STRIPE, STRIPE)])
    pltpu.sync_copy(ones_hbm, ones_v)
    pltpu.sync_copy(dst_hbm.at[pl.ds(wid * EPW, CHUNK)], idxd0)
    plsc.subcore_barrier()
    for k in range(NCHUNK):
        cur = k % 2
        if k + 1 < NCHUNK:
            nxt = (k + 1) % 2
            if scps[nxt] is not None:
                scps[nxt].wait()
                scps[nxt] = None
            pltpu.sync_copy(
                dst_hbm.at[pl.ds(wid * EPW + (k + 1) * CHUNK, CHUNK)],
                idxd[nxt])
        scps[cur] = pltpu.async_copy(ones_v, deg_sh.at[idxd[cur]],
                                     ssems[cur], add=True)
    for b in range(2):
        if scps[b] is not None:
            scps[b].wait()
    plsc.subcore_barrier()
    pltpu.sync_copy(deg_sh.at[pl.ds(sid * STRIPE, STRIPE)], dn_v)

    def rep_row(i, _):
        chunk = dn_v[pl.ds(i * 16, 16)]
        for m in range(16):
            rep_v[i * 16 + m] = jnp.full((16,), chunk[m], jnp.float32)
        return 0

    lax.fori_loop(0, STRIPE // 16, rep_row, 0)
    pltpu.sync_copy(rep_v, out_hbm.at[cid, pl.ds(sid * STRIPE, STRIPE)])


_deg_scatter = pl.kernel(
    _deg_body,
    mesh=_mesh,
    compiler_params=pltpu.CompilerParams(use_tc_tiling_on_sc=False),
    out_type=jax.ShapeDtypeStruct((NC, N_PAD, H), jnp.float32),
    scratch_types=[
        pltpu.VMEM((CHUNK,), jnp.int32),
        pltpu.VMEM((CHUNK,), jnp.int32),
        pltpu.VMEM((CHUNK,), jnp.float32),
        pltpu.VMEM((STRIPE,), jnp.float32),
        pltpu.VMEM((STRIPE, H), jnp.float32),
        pltpu.VMEM_SHARED((N_PAD,), jnp.float32),
        pltpu.SemaphoreType.DMA,
        pltpu.SemaphoreType.DMA,
    ],
)


def _make_edge_scatter(gather: bool):
    def body(src_hbm, dst_hbm, table_hbm, zeros_hbm, out_hbm,
             idxs0, idxs1, idxd0, idxd1, rows0, rows1, acc_sh,
             sem0, sem1, ssem0, ssem1):
        cid = lax.axis_index("c")
        sid = lax.axis_index("s")
        wid = cid * NS + sid
        idxs = [idxs0, idxs1]
        idxd = [idxd0, idxd1]
        rows = [rows0, rows1]
        sems = [sem0, sem1]
        ssems = [ssem0, ssem1]
        cps = [None, None]
        scps = [None, None]
        pltpu.sync_copy(zeros_hbm, acc_sh.at[pl.ds(sid * STRIPE, STRIPE)])
        if gather:
            pltpu.sync_copy(src_hbm.at[pl.ds(wid * EPW, CHUNK)], idxs0)
            cps[0] = pltpu.async_copy(table_hbm.at[idxs0], rows0, sem0)
            pltpu.sync_copy(dst_hbm.at[pl.ds(wid * EPW, CHUNK)], idxd0)
        else:
            pltpu.sync_copy(table_hbm, rows0)
        plsc.subcore_barrier()
        for k in range(NCHUNK):
            base = wid * EPW + k * CHUNK
            cur = k % 2
            if gather:
                if k + 1 < NCHUNK:
                    nxt = (k + 1) % 2
                    pltpu.sync_copy(src_hbm.at[pl.ds(base + CHUNK, CHUNK)],
                                    idxs[nxt])
                    if scps[nxt] is not None:
                        scps[nxt].wait()
                        scps[nxt] = None
                    cps[nxt] = pltpu.async_copy(table_hbm.at[idxs[nxt]],
                                                rows[nxt], sems[nxt])
                    pltpu.sync_copy(dst_hbm.at[pl.ds(base + CHUNK, CHUNK)],
                                    idxd[nxt])
                cps[cur].wait()
                scps[cur] = pltpu.async_copy(rows[cur], acc_sh.at[idxd[cur]],
                                             ssems[cur], add=True)
            else:
                pltpu.sync_copy(dst_hbm.at[pl.ds(base, CHUNK)], idxd0)
                pltpu.sync_copy(rows0, acc_sh.at[idxd0], add=True)
        for b in range(2):
            if scps[b] is not None:
                scps[b].wait()
        plsc.subcore_barrier()
        pltpu.sync_copy(acc_sh.at[pl.ds(sid * STRIPE, STRIPE)],
                        out_hbm.at[cid, pl.ds(sid * STRIPE, STRIPE)])

    return pl.kernel(
        body,
        mesh=_mesh,
        compiler_params=pltpu.CompilerParams(use_tc_tiling_on_sc=False),
        out_type=jax.ShapeDtypeStruct((NC, N_PAD, H), jnp.float32),
        scratch_types=[
            pltpu.VMEM((CHUNK,), jnp.int32),
            pltpu.VMEM((CHUNK,), jnp.int32),
            pltpu.VMEM((CHUNK,), jnp.int32),
            pltpu.VMEM((CHUNK,), jnp.int32),
            pltpu.VMEM((CHUNK, H), jnp.float32),
            pltpu.VMEM((CHUNK, H), jnp.float32),
            pltpu.VMEM_SHARED((N_PAD, H), jnp.float32),
            pltpu.SemaphoreType.DMA,
            pltpu.SemaphoreType.DMA,
            pltpu.SemaphoreType.DMA,
            pltpu.SemaphoreType.DMA,
        ],
    )


_edge_scatter = _make_edge_scatter(gather=True)


RW = N // 8
RWP = N_PAD // 8
GRP = 128 // H

_LANE = np.arange(128)
_P_BCAST = ((_LANE[:, None] % H == 0)
            & (_LANE[:, None] // H == _LANE[None, :] // H)).astype(np.float32)
_G_SUM = (_LANE[:, None] // H == _LANE[None, :] // H).astype(np.float32)


_SPB = E // 10


def _split_body(e_ref, s_ref, d_ref):
    s_ref[...] = e_ref[0].reshape(1, _SPB // 128, 128)
    d_ref[...] = e_ref[1].reshape(1, _SPB // 128, 128)


_tcsplit = pl.pallas_call(
    _split_body,
    grid=(10,),
    in_specs=[pl.BlockSpec((2, _SPB), lambda i: (0, i))],
    out_specs=(pl.BlockSpec((1, _SPB // 128, 128), lambda i: (i, 0, 0)),
               pl.BlockSpec((1, _SPB // 128, 128), lambda i: (i, 0, 0))),
    out_shape=(jax.ShapeDtypeStruct((10, _SPB // 128, 128), jnp.int32),
               jax.ShapeDtypeStruct((10, _SPB // 128, 128), jnp.int32)),
)


def _tc1a_body(x3_ref, ke3_ref, w1a_ref, w1b_ref, hraw_ref):
    h3 = lax.dot_general(x3_ref[...], w1a_ref[...],
                         dimension_numbers=(((2,), (0,)), ((), ())),
                         preferred_element_type=jnp.float32)
    k3 = lax.dot_general(ke3_ref[...], w1b_ref[...],
                         dimension_numbers=(((2,), (0,)), ((), ())),
                         preferred_element_type=jnp.float32)
    hraw_ref[...] = (h3 + k3).reshape(RW, 128)


_tc1a = pl.pallas_call(
    _tc1a_body,
    out_shape=jax.ShapeDtypeStruct((RW, 128), jnp.float32),
)


def _tc1b_body(deg_ref, hraw_ref, g1_ref, dinv_ref):
    deg = 1.0 + deg_ref[0, :RW] + deg_ref[1, :RW]
    dinv = lax.rsqrt(deg)
    g1_ref[...] = hraw_ref[...] * dinv
    dinv_ref[...] = dinv


_tc1b = pl.pallas_call(
    _tc1b_body,
    out_shape=(jax.ShapeDtypeStruct((RW, 128), jnp.float32),
               jax.ShapeDtypeStruct((RW, 128), jnp.float32)),
)


def _tc2_body(acc_ref, g1_ref, dinv_ref, b1_ref, w2blk_ref, g2_ref):
    acc = acc_ref[0, :RW] + acc_ref[1, :RW] + g1_ref[...]
    h1 = jnp.maximum(acc * dinv_ref[...] + b1_ref[...], 0.0)
    g2_ref[...] = jnp.dot(h1, w2blk_ref[...],
                          preferred_element_type=jnp.float32) * dinv_ref[...]


_tc2 = pl.pallas_call(
    _tc2_body,
    out_shape=jax.ShapeDtypeStruct((RW, 128), jnp.float32),
)


def _tc3_body(acc_ref, g2_ref, dinv_ref, b2_ref, p_ref, g_ref, out_ref):
    t = (acc_ref[0, :RW] + acc_ref[1, :RW] + g2_ref[...]) * dinv_ref[...]
    t = t + b2_ref[...]
    lane = lax.broadcasted_iota(jnp.int32, (RW, 128), 1)
    jm = lane % H
    valid = jm < C
    tm = jnp.where(valid, t, -3.0e38)
    for s in (1, 2, 4, 8):
        rolled = jnp.concatenate([tm[:, s:], tm[:, :s]], axis=1)
        keep = (jm + s) < H
        tm = jnp.where(keep, jnp.maximum(tm, rolled), tm)
    mb = jnp.dot(tm, p_ref[...], preferred_element_type=jnp.float32)
    ex = jnp.where(valid, jnp.exp(t - mb), 0.0)
    ssum = jnp.dot(ex, g_ref[...], preferred_element_type=jnp.float32)
    out_ref[...] = t - mb - jnp.log(ssum)


_tc3 = pl.pallas_call(
    _tc3_body,
    out_shape=jax.ShapeDtypeStruct((RW, 128), jnp.float32),
)


def kernel(x, edge_index, ke, W1, b1, W2, b2):
    src2, dst2 = _tcsplit(edge_index)
    src = src2.reshape(E)
    dst = dst2.reshape(E)
    zeros_t = jnp.asarray(_ZEROS_T)
    eye8 = jnp.eye(GRP, dtype=jnp.float32)
    W2p = jnp.concatenate([W2, jnp.zeros((H, H - C), W2.dtype)], axis=1)
    W2blk = jnp.kron(eye8, W2p)
    b1t = jnp.tile(b1, GRP)
    b2t = jnp.tile(jnp.concatenate([b2, jnp.zeros((H - C,), b2.dtype)]), GRP)
    x3 = x.reshape(RW, GRP, D)
    ke3 = ke.reshape(RW, GRP, KE)

    deg_parts = _deg_scatter(dst, jnp.asarray(_ONES_N), jnp.asarray(_ZEROS_N))
    hraw = _tc1a(x3, ke3, W1[:D], W1[D:])
    g1w, dinvw = _tc1b(deg_parts.reshape(NC, RWP, 128), hraw)
    acc1 = _edge_scatter(src, dst, g1w.reshape(N, H), zeros_t)
    g2w = _tc2(acc1.reshape(NC, RWP, 128), g1w, dinvw, b1t, W2blk)
    acc2 = _edge_scatter(src, dst, g2w.reshape(N, H), zeros_t)
    outw = _tc3(acc2.reshape(NC, RWP, 128), g2w, dinvw, b2t,
                jnp.asarray(_P_BCAST), jnp.asarray(_G_SUM))
    return outw.reshape(N, H)[:, :C]
